# baseline (speedup 1.0000x reference)
import functools

import jax
import jax.numpy as jnp
from jax import lax
from jax.experimental import pallas as pl
from jax.experimental.pallas import tpu as pltpu
from jax.experimental.pallas import tpu_sc as plsc

B, C, N, K = 16, 96, 1024, 16
O = 96
D = 128
NW = 32
BH = B // 2
S = BH * N // NW
CH = 8
NCH = S // CH
G = CH * K


def _tc_tables_body(x_ref, e_ref, w_ref, b_ref, y1_ref, y2_ref, idx_ref):
    xb = x_ref[0]
    w = w_ref[...]
    w1 = w[:, :C]
    w2 = w[:, C:]
    dn = (((0,), (1,)), ((), ()))
    y1 = lax.dot_general(xb, w1 - w2, dn, preferred_element_type=jnp.float32)
    y1 = y1 + b_ref[...][None, :]
    y2 = lax.dot_general(xb, w2, dn, preferred_element_type=jnp.float32)
    pad = jnp.zeros((N, D - O), jnp.float32)
    y1_ref[0] = jnp.concatenate([y1, pad], axis=1)
    y2_ref[0] = jnp.concatenate([y2, pad], axis=1)
    @pl.when(pl.program_id(0) == 0)
    def _():
        offs = lax.broadcasted_iota(jnp.int32, (2, BH, N * K), 1) * N
        idx_ref[...] = e_ref[...] + offs


def _tc_tables(x3, e3, W, b):
    return pl.pallas_call(
        _tc_tables_body,
        grid=(BH,),
        in_specs=[
            pl.BlockSpec((1, C, N), lambda i: (i, 0, 0)),
            pl.BlockSpec((2, BH, N * K), lambda i: (0, 0, 0)),
            pl.BlockSpec((O, 2 * C), lambda i: (0, 0)),
            pl.BlockSpec((O,), lambda i: (0,)),
        ],
        out_specs=[
            pl.BlockSpec((1, N, D), lambda i: (i, 0, 0)),
            pl.BlockSpec((1, N, D), lambda i: (i, 0, 0)),
            pl.BlockSpec((2, BH, N * K), lambda i: (0, 0, 0)),
        ],
        out_shape=[
            jax.ShapeDtypeStruct((BH, N, D), jnp.float32),
            jax.ShapeDtypeStruct((BH, N, D), jnp.float32),
            jax.ShapeDtypeStruct((2, BH, N * K), jnp.int32),
        ],
    )(x3, e3, W, b)


def _sc_body(tab1_hbm, tab0_hbm, idx1_hbm, idx0_hbm, out_hbm,
             i1_v, i0_v, r1_v, r0_v, o_v, sem0, sem1, osem0, osem1):
    wid = lax.axis_index("s") * 2 + lax.axis_index("c")
    base = wid * S
    crow = wid * NCH
    sems = (sem0, sem1)
    osems = (osem0, osem1)

    pltpu.sync_copy(idx1_hbm.at[pl.ds(crow, NCH)], i1_v)
    pltpu.sync_copy(idx0_hbm.at[pl.ds(crow, NCH)], i0_v)

    def fire(c, sl):
        pltpu.async_copy(tab1_hbm.at[i1_v.at[c]], r1_v.at[sl], sems[sl])
        pltpu.async_copy(tab0_hbm.at[i0_v.at[c]], r0_v.at[sl], sems[sl])

    def wait_slot(c, sl):
        pltpu.make_async_copy(tab1_hbm.at[i1_v.at[c]], r1_v.at[sl],
                              sems[sl]).wait()
        pltpu.make_async_copy(tab0_hbm.at[i0_v.at[c]], r0_v.at[sl],
                              sems[sl]).wait()

    def out_rows(c):
        return out_hbm.at[pl.ds(base + c * CH, CH)]

    def compute_slot(c, sl):
        @pl.when(c >= 2)
        def _():
            pltpu.make_async_copy(o_v.at[sl], out_rows(c - 2),
                                  osems[sl]).wait()

        def node_body(i, c2):
            r0 = i * K
            for cg in range(O // 16):
                s = pl.ds(cg * 16, 16)
                acc = r1_v[sl, r0, s] + r0_v[sl, r0, s]
                for k in range(1, K):
                    acc = jnp.maximum(
                        acc, r1_v[sl, r0 + k, s] + r0_v[sl, r0 + k, s])
                o_v[sl, i, s] = jnp.maximum(acc, 0.0)
            return c2

        lax.fori_loop(0, CH, node_body, 0)
        pltpu.async_copy(o_v.at[sl], out_rows(c), osems[sl])

    fire(0, 0)

    def outer_body(cc, carry):
        for bslot in range(2):
            c = cc + bslot

            @pl.when(c + 1 < NCH)
            def _():
                fire(c + 1, (bslot + 1) % 2)

            wait_slot(c, bslot)
            compute_slot(c, bslot)
        return carry

    lax.fori_loop(0, NCH // 2, lambda t, carry: outer_body(t * 2, carry), 0)
    pltpu.make_async_copy(o_v.at[0], out_rows(NCH - 2), osems[0]).wait()
    pltpu.make_async_copy(o_v.at[1], out_rows(NCH - 1), osems[1]).wait()


def _sc_gather_max(tab1, tab0, idx1, idx0):
    kfn = functools.partial(
        pl.kernel,
        mesh=plsc.VectorSubcoreMesh(core_axis_name="c", subcore_axis_name="s"),
        out_type=jax.ShapeDtypeStruct((BH * N, D), jnp.float32),
        scratch_types=[
            pltpu.VMEM((NCH, G), jnp.int32),
            pltpu.VMEM((NCH, G), jnp.int32),
            pltpu.VMEM((2, G, D), jnp.float32),
            pltpu.VMEM((2, G, D), jnp.float32),
            pltpu.VMEM((2, CH, D), jnp.float32),
            pltpu.SemaphoreType.DMA,
            pltpu.SemaphoreType.DMA,
            pltpu.SemaphoreType.DMA,
            pltpu.SemaphoreType.DMA,
        ],
    )(_sc_body)
    return kfn(tab1, tab0, idx1, idx0)


def kernel(x, edge_index, W, b):
    x3 = x.reshape(B, C, N)
    e3 = edge_index.reshape(2, B, N * K)
    halves = []
    for h in range(2):
        xh = x3[h * BH:(h + 1) * BH]
        eh = e3[:, h * BH:(h + 1) * BH]
        y1, y2, idxr = _tc_tables(xh, eh, W, b)
        tab1 = y1.reshape(BH * N, D)
        tab0 = y2.reshape(BH * N, D)
        idx1 = idxr[1].reshape(BH * N * K // G, G)
        idx0 = idxr[0].reshape(BH * N * K // G, G)
        out = _sc_gather_max(tab1, tab0, idx1, idx0)
        out = out.reshape(BH, N, D)[:, :, :O]
        halves.append(out.transpose(0, 2, 1).reshape(BH, O, N, 1))
    return jnp.concatenate(halves, axis=0)


# speedup vs baseline: 1.0297x; 1.0297x over previous
import functools

import jax
import jax.numpy as jnp
from jax import lax
from jax.experimental import pallas as pl
from jax.experimental.pallas import tpu as pltpu
from jax.experimental.pallas import tpu_sc as plsc

B, C, N, K = 16, 96, 1024, 16
O = 96
D = 128
NW = 32
S = B * N // NW
CH = 8
NCH = S // CH
G = CH * K
NCB = N * K // G


def _tc_tables_body(x_ref, e_ref, w_ref, b_ref, y1_ref, y2_ref, idx_ref):
    xb = x_ref[0]
    w = w_ref[...]
    w1 = w[:, :C]
    w2 = w[:, C:]
    dn = (((0,), (1,)), ((), ()))
    y1 = lax.dot_general(xb, w1 - w2, dn, preferred_element_type=jnp.float32)
    y1 = y1 + b_ref[...][None, :]
    y2 = lax.dot_general(xb, w2, dn, preferred_element_type=jnp.float32)
    pad = jnp.zeros((N, D - O), jnp.float32)
    y1_ref[0] = jnp.concatenate([y1, pad], axis=1)
    y2_ref[0] = jnp.concatenate([y2, pad], axis=1)
    @pl.when(pl.program_id(0) == 0)
    def _():
        offs = lax.broadcasted_iota(jnp.int32, (2, B, N * K), 1) * N
        idx_ref[...] = e_ref[...] + offs


def _tc_tables(x3, e3, W, b):
    return pl.pallas_call(
        _tc_tables_body,
        grid=(B,),
        in_specs=[
            pl.BlockSpec((1, C, N), lambda i: (i, 0, 0)),
            pl.BlockSpec((2, B, N * K), lambda i: (0, 0, 0)),
            pl.BlockSpec((O, 2 * C), lambda i: (0, 0)),
            pl.BlockSpec((O,), lambda i: (0,)),
        ],
        out_specs=[
            pl.BlockSpec((1, N, D), lambda i: (i, 0, 0)),
            pl.BlockSpec((1, N, D), lambda i: (i, 0, 0)),
            pl.BlockSpec((2, B, N * K), lambda i: (0, 0, 0)),
        ],
        out_shape=[
            jax.ShapeDtypeStruct((B, N, D), jnp.float32),
            jax.ShapeDtypeStruct((B, N, D), jnp.float32),
            jax.ShapeDtypeStruct((2, B, N * K), jnp.int32),
        ],
    )(x3, e3, W, b)


def _sc_body(tab1_hbm, tab0_hbm, idx1_hbm, idx0_hbm, out_hbm,
             i1_v, i0_v, r1_v, r0_v, o_v, sem0, sem1, osem0, osem1):
    wid = lax.axis_index("s") * 2 + lax.axis_index("c")
    base = wid * S
    crow = wid * NCH
    sems = (sem0, sem1)
    osems = (osem0, osem1)

    pltpu.sync_copy(idx1_hbm.at[pl.ds(crow, NCH)], i1_v)
    pltpu.sync_copy(idx0_hbm.at[pl.ds(crow, NCH)], i0_v)

    def fire(c, sl):
        pltpu.async_copy(tab1_hbm.at[i1_v.at[c]], r1_v.at[sl], sems[sl])
        pltpu.async_copy(tab0_hbm.at[i0_v.at[c]], r0_v.at[sl], sems[sl])

    def wait_slot(c, sl):
        pltpu.make_async_copy(tab1_hbm.at[i1_v.at[c]], r1_v.at[sl],
                              sems[sl]).wait()
        pltpu.make_async_copy(tab0_hbm.at[i0_v.at[c]], r0_v.at[sl],
                              sems[sl]).wait()

    def out_rows(c):
        return out_hbm.at[pl.ds(base + c * CH, CH)]

    def compute_slot(c, sl):
        @pl.when(c >= 2)
        def _():
            pltpu.make_async_copy(o_v.at[sl], out_rows(c - 2),
                                  osems[sl]).wait()

        def node_body(i, c2):
            r0 = i * K
            for cg in range(O // 16):
                s = pl.ds(cg * 16, 16)
                acc = r1_v[sl, r0, s] + r0_v[sl, r0, s]
                for k in range(1, K):
                    acc = jnp.maximum(
                        acc, r1_v[sl, r0 + k, s] + r0_v[sl, r0 + k, s])
                o_v[sl, i, s] = jnp.maximum(acc, 0.0)
            return c2

        lax.fori_loop(0, CH, node_body, 0)
        pltpu.async_copy(o_v.at[sl], out_rows(c), osems[sl])

    fire(0, 0)

    def outer_body(cc, carry):
        for bslot in range(2):
            c = cc + bslot

            @pl.when(c + 1 < NCH)
            def _():
                fire(c + 1, (bslot + 1) % 2)

            wait_slot(c, bslot)
            compute_slot(c, bslot)
        return carry

    lax.fori_loop(0, NCH // 2, lambda t, carry: outer_body(t * 2, carry), 0)
    pltpu.make_async_copy(o_v.at[0], out_rows(NCH - 2), osems[0]).wait()
    pltpu.make_async_copy(o_v.at[1], out_rows(NCH - 1), osems[1]).wait()


def _sc_gather_max(tab1, tab0, idx1, idx0):
    kfn = functools.partial(
        pl.kernel,
        mesh=plsc.VectorSubcoreMesh(core_axis_name="c", subcore_axis_name="s"),
        out_type=jax.ShapeDtypeStruct((B * N, D), jnp.float32),
        scratch_types=[
            pltpu.VMEM((NCH, G), jnp.int32),
            pltpu.VMEM((NCH, G), jnp.int32),
            pltpu.VMEM((2, G, D), jnp.float32),
            pltpu.VMEM((2, G, D), jnp.float32),
            pltpu.VMEM((2, CH, D), jnp.float32),
            pltpu.SemaphoreType.DMA,
            pltpu.SemaphoreType.DMA,
            pltpu.SemaphoreType.DMA,
            pltpu.SemaphoreType.DMA,
        ],
    )(_sc_body)
    return kfn(tab1, tab0, idx1, idx0)


def kernel(x, edge_index, W, b):
    x3 = x.reshape(B, C, N)
    e3 = edge_index.reshape(2, B, N * K)
    y1, y2, idxr = _tc_tables(x3, e3, W, b)
    tab1 = y1.reshape(B * N, D)
    tab0 = y2.reshape(B * N, D)
    idx1 = idxr[1].reshape(B * N * K // G, G)
    idx0 = idxr[0].reshape(B * N * K // G, G)
    out = _sc_gather_max(tab1, tab0, idx1, idx0)
    out = out.reshape(B, N, D)[:, :, :O]
    return out.transpose(0, 2, 1).reshape(B, O, N, 1)
